# baseline (speedup 1.0000x reference)
import jax
import jax.numpy as jnp
from jax import lax
from jax.experimental import pallas as pl
from jax.experimental.pallas import tpu as pltpu
from jax.experimental.pallas import tpu_sc as plsc

NUM_CLASSES = 10000
N_ROWS = 320000
D_COLS = 128
NC = 2
NS = 16
BLK = 64
NBLK = N_ROWS // BLK
BLKS_PER_TILE = NBLK // NS
EXTRA = NBLK - BLKS_PER_TILE * NS
CPT = NUM_CLASSES // NS
HALF = D_COLS // NC
W2 = HALF + 16
RING = 4


def _sc_body(x_hbm, lab_hbm, numw_hbm, miu_hbm, std_hbm,
             acc1, acc2, x0, x1, x2, x3, q0, q1, q2, q3, idxb,
             is0, is1, is2, is3, ss0, ss1, ss2, ss3):
    cid = lax.axis_index("c")
    sid = lax.axis_index("s")
    c0 = cid * HALF
    xs = (x0, x1, x2, x3)
    qs = (q0, q1, q2, q3)
    isems = (is0, is1, is2, is3)
    ssems = (ss0, ss1, ss2, ss3)

    def xslice(b):
        return x_hbm.at[pl.ds(b * BLK, BLK), pl.ds(c0, HALF)]

    zeros16 = jnp.zeros((16,), jnp.float32)
    ones16 = jnp.ones((16,), jnp.float32)

    @pl.loop(0, BLK)
    def _(i):
        for c4 in range(HALF // 16):
            x0[i, pl.ds(c4 * 16, 16)] = zeros16
        for c4 in range(W2 // 16):
            q0[i, pl.ds(c4 * 16, 16)] = zeros16

    base = sid * CPT
    for j in range(10):
        n = CPT - 9 * 64 if j == 9 else 64
        pltpu.sync_copy(x0.at[pl.ds(0, n), :],
                        acc1.at[pl.ds(base + j * 64, n), :])
        pltpu.sync_copy(q0.at[pl.ds(0, n), :],
                        acc2.at[pl.ds(base + j * 64, n), :])

    @pl.loop(0, BLK)
    def _(i):
        for q in qs:
            q[i, pl.ds(HALF, 16)] = ones16

    for par in range(RING - 1):
        pltpu.async_copy(xslice(sid + par * NS), xs[par], isems[par])
        pltpu.async_copy(lab_hbm.at[sid + par * NS], idxb.at[par],
                         isems[par])

    plsc.subcore_barrier()

    def square(xv, qv):
        @pl.loop(0, BLK, step=4)
        def _(i):
            for r in range(4):
                for c4 in range(HALF // 16):
                    v = xv[i + r, pl.ds(c4 * 16, 16)]
                    qv[i + r, pl.ds(c4 * 16, 16)] = v * v

    def refill(par, b):
        pltpu.async_copy(xslice(b), xs[par], isems[par])
        pltpu.async_copy(lab_hbm.at[b], idxb.at[par], isems[par])

    def wait_in(par):
        pltpu.make_async_copy(xslice(sid), xs[par], isems[par]).wait()
        pltpu.make_async_copy(lab_hbm.at[sid], idxb.at[par],
                              isems[par]).wait()

    def drain_scatter(par):
        pltpu.make_async_copy(xs[par], acc1.at[idxb.at[par]],
                              ssems[par]).wait()
        pltpu.make_async_copy(qs[par], acc2.at[idxb.at[par]],
                              ssems[par]).wait()

    @pl.loop(0, BLKS_PER_TILE, step=RING)
    def _(k):
        for p in range(RING):
            par = p % RING
            nxt = (p + RING - 1) % RING
            idx = idxb.at[par]
            wait_in(par)
            pltpu.async_copy(xs[par], acc1.at[idx], ssems[par], add=True)
            square(xs[par], qs[par])
            pltpu.async_copy(qs[par], acc2.at[idx], ssems[par], add=True)

            if p == 0:
                @pl.when(k >= 1)
                def _():
                    drain_scatter(nxt)

                refill(nxt, sid + (k + RING - 1) * NS)
            else:
                drain_scatter(nxt)

                @pl.when(k + p + RING - 1 < BLKS_PER_TILE)
                def _():
                    refill(nxt, sid + (k + p + RING - 1) * NS)

    drain_scatter(RING - 1)

    @pl.when(sid < EXTRA)
    def _():
        b = BLKS_PER_TILE * NS + sid
        idx = idxb.at[0]
        pltpu.sync_copy(xslice(b), x0)
        pltpu.sync_copy(lab_hbm.at[b], idx)
        pltpu.sync_copy(x0, acc1.at[idx], add=True)
        square(x0, q0)
        pltpu.sync_copy(q0, acc2.at[idx], add=True)

    plsc.subcore_barrier()

    rows = pl.ds(base, CPT)
    pltpu.sync_copy(acc1.at[rows, :], miu_hbm.at[rows, pl.ds(c0, HALF)])
    pltpu.sync_copy(acc2.at[rows, pl.ds(0, HALF)],
                    std_hbm.at[rows, pl.ds(c0, HALF)])

    @pl.when(cid == 0)
    def _():
        pltpu.sync_copy(acc2.at[rows, pl.ds(HALF, 16)],
                        numw_hbm.at[rows, :])


@jax.jit
def _tracker(X, labels2d):
    mesh = plsc.VectorSubcoreMesh(core_axis_name="c", subcore_axis_name="s")
    f = pl.kernel(
        _sc_body,
        compiler_params=pltpu.CompilerParams(use_tc_tiling_on_sc=False),
        out_type=(
            jax.ShapeDtypeStruct((NUM_CLASSES, 16), jnp.float32),
            jax.ShapeDtypeStruct((NUM_CLASSES, D_COLS), jnp.float32),
            jax.ShapeDtypeStruct((NUM_CLASSES, D_COLS), jnp.float32),
        ),
        mesh=mesh,
        scratch_types=[
            pltpu.VMEM_SHARED((NUM_CLASSES, HALF), jnp.float32),
            pltpu.VMEM_SHARED((NUM_CLASSES, W2), jnp.float32),
            pltpu.VMEM((BLK, HALF), jnp.float32),
            pltpu.VMEM((BLK, HALF), jnp.float32),
            pltpu.VMEM((BLK, HALF), jnp.float32),
            pltpu.VMEM((BLK, HALF), jnp.float32),
            pltpu.VMEM((BLK, W2), jnp.float32),
            pltpu.VMEM((BLK, W2), jnp.float32),
            pltpu.VMEM((BLK, W2), jnp.float32),
            pltpu.VMEM((BLK, W2), jnp.float32),
            pltpu.VMEM((RING, BLK), jnp.int32),
            pltpu.SemaphoreType.DMA,
            pltpu.SemaphoreType.DMA,
            pltpu.SemaphoreType.DMA,
            pltpu.SemaphoreType.DMA,
            pltpu.SemaphoreType.DMA,
            pltpu.SemaphoreType.DMA,
            pltpu.SemaphoreType.DMA,
            pltpu.SemaphoreType.DMA,
        ],
    )
    return f(X, labels2d)


def kernel(X, labels):
    labels2d = labels.astype(jnp.int32).reshape(N_ROWS // BLK, BLK)
    numw, miu, std = _tracker(X, labels2d)
    return (numw[:, :1], miu, std)


# speedup vs baseline: 1.8852x; 1.8852x over previous
import jax
import jax.numpy as jnp
from jax import lax
from jax.experimental import pallas as pl
from jax.experimental.pallas import tpu as pltpu
from jax.experimental.pallas import tpu_sc as plsc

NUM_CLASSES = 10000
N_ROWS = 320000
D_COLS = 128
NC = 2
NS = 16
BLK = 128
NBLK = N_ROWS // BLK
BLKS_PER_TILE = NBLK // NS
EXTRA = NBLK - BLKS_PER_TILE * NS
CPT = NUM_CLASSES // NS
HALF = D_COLS // NC
CROWS = 640


def _sc_body(x_hbm, lab_hbm, numw_hbm, miu_hbm, std_hbm,
             miu_sh, std_sh, numr_sh, xa, xb_, sqa, sqb_, idxb, cnt, zbuf,
             riota, isem_a, isem_b, ssem_a, ssem_b):
    cid = lax.axis_index("c")
    sid = lax.axis_index("s")
    c0 = cid * HALF
    bufs = ((xa, sqa, isem_a, ssem_a), (xb_, sqb_, isem_b, ssem_b))

    def xslice(b):
        return x_hbm.at[pl.ds(b * BLK, BLK), pl.ds(c0, HALF)]

    for par in range(2):
        xv, _, isem, _ = bufs[par]
        pltpu.async_copy(xslice(sid + par * NS), xv, isem)
        pltpu.async_copy(lab_hbm.at[sid + par * NS], idxb.at[par], isem)

    zeros16 = jnp.zeros((16,), jnp.float32)

    @pl.loop(0, 64)
    def _(i):
        for c4 in range(HALF // 16):
            zbuf[i, pl.ds(c4 * 16, 16)] = zeros16

    base = sid * CPT
    for off, n in ((0, 64), (64, 64), (128, 64), (192, 64), (256, 64),
                   (320, 64), (384, 64), (448, 64), (512, 64), (576, 49)):
        pltpu.sync_copy(zbuf.at[pl.ds(0, n), :],
                        miu_sh.at[pl.ds(base + off, n), :])
        pltpu.sync_copy(zbuf.at[pl.ds(0, n), :],
                        std_sh.at[pl.ds(base + off, n), :])

    @pl.when(cid == 0)
    def _():
        @pl.loop(0, CROWS)
        def _(i):
            cnt[i, pl.ds(0, 16)] = zeros16

        iota16 = lax.iota(jnp.int32, 16)

        @pl.loop(0, 5)
        def _(j):
            for g in range(8):
                riota[j, pl.ds(g * 16, 16)] = iota16 + j * 128 + g * 16

        @pl.when(sid == 0)
        def _():
            for j in range(10):
                pltpu.sync_copy(zbuf.at[pl.ds(0, 64), pl.ds(0, 16)],
                                numr_sh.at[pl.ds(j * 64, 64), :])

    plsc.subcore_barrier()

    def square(src, dst):
        @pl.loop(0, BLK, step=4)
        def _(i):
            for r in range(4):
                for c4 in range(HALF // 16):
                    v = src[i + r, pl.ds(c4 * 16, 16)]
                    dst[i + r, pl.ds(c4 * 16, 16)] = v * v

    ones16 = jnp.ones((16,), jnp.float32)

    def count_block(par):
        @pl.when(cid == 0)
        def _():
            for g in range(BLK // 16):
                labv = idxb[par, pl.ds(g * 16, 16)]
                plsc.addupdate_scatter(
                    cnt, [labv >> 4, labv & 15], ones16)

    @pl.loop(0, BLKS_PER_TILE, step=2)
    def _(k):
        for par in range(2):
            kk = k + par
            xv, sqv, isem, ssem = bufs[par]
            idx = idxb.at[par]
            pltpu.make_async_copy(xslice(sid), xv, isem).wait()
            pltpu.make_async_copy(lab_hbm.at[sid], idx, isem).wait()
            cp_miu = pltpu.async_copy(xv, miu_sh.at[idx], ssem, add=True)
            square(xv, sqv)
            cp_std = pltpu.async_copy(sqv, std_sh.at[idx], ssem, add=True)
            count_block(par)
            cp_miu.wait()
            cp_std.wait()

            @pl.when(kk + 2 < BLKS_PER_TILE)
            def _():
                b_next = sid + (kk + 2) * NS
                pltpu.async_copy(xslice(b_next), xv, isem)
                pltpu.async_copy(lab_hbm.at[b_next], idx, isem)

    @pl.when(sid < EXTRA)
    def _():
        b = BLKS_PER_TILE * NS + sid
        xv, sqv, _, _ = bufs[0]
        idx = idxb.at[0]
        pltpu.sync_copy(xslice(b), xv)
        pltpu.sync_copy(lab_hbm.at[b], idx)
        pltpu.sync_copy(xv, miu_sh.at[idx], add=True)
        square(xv, sqv)
        pltpu.sync_copy(sqv, std_sh.at[idx], add=True)
        count_block(0)

    @pl.when(cid == 0)
    def _():
        for j in range(5):
            pltpu.sync_copy(cnt.at[pl.ds(j * 128, 128), :],
                            numr_sh.at[riota.at[j]], add=True)

    plsc.subcore_barrier()

    rows = pl.ds(base, CPT)
    pltpu.sync_copy(miu_sh.at[rows, :], miu_hbm.at[rows, pl.ds(c0, HALF)])
    pltpu.sync_copy(std_sh.at[rows, :], std_hbm.at[rows, pl.ds(c0, HALF)])

    @pl.when((cid == 0) & (sid == 0))
    def _():
        pltpu.sync_copy(numr_sh, numw_hbm)


@jax.jit
def _tracker(X, labels2d):
    mesh = plsc.VectorSubcoreMesh(core_axis_name="c", subcore_axis_name="s")
    f = pl.kernel(
        _sc_body,
        compiler_params=pltpu.CompilerParams(use_tc_tiling_on_sc=False,
                                             needs_layout_passes=False),
        out_type=(
            jax.ShapeDtypeStruct((CROWS, 16), jnp.float32),
            jax.ShapeDtypeStruct((NUM_CLASSES, D_COLS), jnp.float32),
            jax.ShapeDtypeStruct((NUM_CLASSES, D_COLS), jnp.float32),
        ),
        mesh=mesh,
        scratch_types=[
            pltpu.VMEM_SHARED((NUM_CLASSES, HALF), jnp.float32),
            pltpu.VMEM_SHARED((NUM_CLASSES, HALF), jnp.float32),
            pltpu.VMEM_SHARED((CROWS, 16), jnp.float32),
            pltpu.VMEM((BLK, HALF), jnp.float32),
            pltpu.VMEM((BLK, HALF), jnp.float32),
            pltpu.VMEM((BLK, HALF), jnp.float32),
            pltpu.VMEM((BLK, HALF), jnp.float32),
            pltpu.VMEM((2, 128), jnp.int32),
            pltpu.VMEM((CROWS, 16), jnp.float32),
            pltpu.VMEM((64, HALF), jnp.float32),
            pltpu.VMEM((5, 128), jnp.int32),
            pltpu.SemaphoreType.DMA,
            pltpu.SemaphoreType.DMA,
            pltpu.SemaphoreType.DMA,
            pltpu.SemaphoreType.DMA,
        ],
    )
    return f(X, labels2d)


def kernel(X, labels):
    labels2d = labels.astype(jnp.int32).reshape(N_ROWS // 128, 128)
    numw, miu, std = _tracker(X, labels2d)
    num = numw.reshape(-1)[:NUM_CLASSES].reshape(NUM_CLASSES, 1)
    return (num, miu, std)
